# Initial kernel scaffold; baseline (speedup 1.0000x reference)
import jax
import jax.numpy as jnp
from jax.experimental import pallas as pl


def kernel(x, W_gate, b_gate, W1, b1, W2, b2):
    raise NotImplementedError("write your pallas kernel here")


# speedup vs baseline: 1.2871x; 1.2871x over previous
import functools

import jax
import jax.numpy as jnp
from jax.experimental import pallas as pl
from jax.experimental.pallas import tpu as pltpu

B, S = 1, 2048
D_MODEL = 1024
D_FF = 2048
E = 8
TOP_K = 2
LANES = 128
NF = 4
FF_T = D_FF // NF


def _moe_body(x_ref, wg_ref, bg_ref, w1_ref, b1_ref, w2_ref, b2_ref,
              out_ref, usage_ref, avg_ref, lbl_ref, c_ref):
    e = pl.program_id(0)
    f = pl.program_id(1)
    lane = jax.lax.broadcasted_iota(jnp.int32, (1, LANES), 1)

    @pl.when(jnp.logical_and(e == 0, f == 0))
    def _router():
        x = x_ref[...]
        logits = jax.lax.dot_general(
            x, wg_ref[...], (((1,), (0,)), ((), ())),
            preferred_element_type=jnp.float32) + bg_ref[...]
        m = jnp.max(logits, axis=1, keepdims=True)
        p = jnp.exp(logits - m)
        p = p / jnp.sum(p, axis=1, keepdims=True)
        p1 = jnp.max(p, axis=1, keepdims=True)
        a1 = jnp.min(jnp.where(p == p1, lane, LANES), axis=1, keepdims=True)
        oh1 = (lane == a1).astype(jnp.float32)
        p_m = jnp.where(lane == a1, -1.0, p)
        p2 = jnp.max(p_m, axis=1, keepdims=True)
        a2 = jnp.min(jnp.where(p_m == p2, lane, LANES), axis=1, keepdims=True)
        oh2 = (lane == a2).astype(jnp.float32)
        wsum = p1 + p2
        c_ref[...] = (p1 / wsum) * oh1 + (p2 / wsum) * oh2
        usage_ref[...] = jnp.sum(oh1 + oh2, axis=0, keepdims=True) / (S * TOP_K)
        avg = jnp.sum(p, axis=0, keepdims=True) / S
        avg_ref[...] = avg
        msk = (lane < E).astype(jnp.float32)
        mean = jnp.sum(avg * msk) / E
        var = jnp.sum(msk * (avg - mean) ** 2) / (E - 1)
        lbl_ref[...] = jnp.full((1, LANES), var, dtype=jnp.float32)
        out_ref[...] = jnp.zeros(out_ref.shape, out_ref.dtype)

    sel = (lane == e).astype(jnp.float32)
    c_col = jnp.sum(c_ref[...] * sel, axis=1, keepdims=True)
    h = jax.lax.dot_general(
        x_ref[...], w1_ref[0], (((1,), (0,)), ((), ())),
        preferred_element_type=jnp.float32) + b1_ref[0]
    h = jnp.maximum(h, 0.0)
    y = jax.lax.dot_general(
        h, w2_ref[0], (((1,), (0,)), ((), ())),
        preferred_element_type=jnp.float32)

    @pl.when(f == 0)
    def _bias2():
        out_ref[...] += c_col * b2_ref[0]

    out_ref[...] += c_col * y


@functools.partial(jax.jit, static_argnames=())
def kernel(x, W_gate, b_gate, W1, b1, W2, b2):
    x2 = x.reshape(S, D_MODEL)
    wg = jnp.zeros((D_MODEL, LANES), jnp.float32).at[:, :E].set(W_gate)
    bg = jnp.full((1, LANES), -1e30, jnp.float32).at[0, :E].set(b_gate)

    grid = (E, NF)
    out, usage, avg, lbl = pl.pallas_call(
        _moe_body,
        grid=grid,
        in_specs=[
            pl.BlockSpec((S, D_MODEL), lambda e, f: (0, 0)),
            pl.BlockSpec((D_MODEL, LANES), lambda e, f: (0, 0)),
            pl.BlockSpec((1, LANES), lambda e, f: (0, 0)),
            pl.BlockSpec((1, D_MODEL, FF_T), lambda e, f: (e, 0, f)),
            pl.BlockSpec((1, 1, FF_T), lambda e, f: (e, 0, f)),
            pl.BlockSpec((1, FF_T, D_MODEL), lambda e, f: (e, f, 0)),
            pl.BlockSpec((1, 1, D_MODEL), lambda e, f: (e, 0, 0)),
        ],
        out_specs=[
            pl.BlockSpec((S, D_MODEL), lambda e, f: (0, 0)),
            pl.BlockSpec((1, LANES), lambda e, f: (0, 0)),
            pl.BlockSpec((1, LANES), lambda e, f: (0, 0)),
            pl.BlockSpec((1, LANES), lambda e, f: (0, 0)),
        ],
        out_shape=[
            jax.ShapeDtypeStruct((S, D_MODEL), jnp.float32),
            jax.ShapeDtypeStruct((1, LANES), jnp.float32),
            jax.ShapeDtypeStruct((1, LANES), jnp.float32),
            jax.ShapeDtypeStruct((1, LANES), jnp.float32),
        ],
        scratch_shapes=[pltpu.VMEM((S, LANES), jnp.float32)],
    )(x2, wg, bg, W1, b1.reshape(E, 1, D_FF), W2, b2.reshape(E, 1, D_MODEL))

    output = out.reshape(B, S, D_MODEL)
    expert_usage = usage[0, :E]
    load_balance_loss = lbl[0, 0]
    return (output, expert_usage, load_balance_loss)
